# Initial kernel scaffold; baseline (speedup 1.0000x reference)
import jax
import jax.numpy as jnp
from jax.experimental import pallas as pl


def kernel(hidden_states, router_W, router_b, Wg, Wu, Wd):
    raise NotImplementedError("write your pallas kernel here")


# speedup vs baseline: 1.4786x; 1.4786x over previous
import functools

import jax
import jax.numpy as jnp
from jax.experimental import pallas as pl
from jax.experimental.pallas import tpu as pltpu

D = 2048
F = 4096
E = 8
T = 2048
BT = 256
MAXB = T // BT + E - 1
PADN = MAXB * BT
BJ = 512
NJ = F // BJ


def _router_body(x_ref, rw_ref, rb_ref, tw_ref, ti_ref):
    l = jnp.dot(x_ref[...], rw_ref[...], preferred_element_type=jnp.float32)
    l = l + rb_ref[...]
    m = jnp.max(l, axis=1, keepdims=True)
    s = jnp.sum(jnp.exp(l - m), axis=1, keepdims=True)
    tw_ref[...] = 1.0 / s
    iota = jax.lax.broadcasted_iota(jnp.int32, l.shape, 1)
    ti_ref[...] = jnp.min(jnp.where(l >= m, iota, E), axis=1, keepdims=True)


def _router(flat, rw, rb):
    return pl.pallas_call(
        _router_body,
        out_shape=(
            jax.ShapeDtypeStruct((T, 1), jnp.float32),
            jax.ShapeDtypeStruct((T, 1), jnp.int32),
        ),
    )(flat, rw, rb.reshape(1, E))


def _moe_body(be_ref, x_ref, wg_ref, wu_ref, wd_ref, tw_ref, o_ref):
    b = pl.program_id(0)
    j = pl.program_id(1)
    active = b < be_ref[MAXB]

    @pl.when(active)
    def _():
        x = x_ref[...]
        g = jnp.dot(x, wg_ref[0], preferred_element_type=jnp.float32)
        u = jnp.dot(x, wu_ref[0], preferred_element_type=jnp.float32)
        h = (jax.nn.silu(g) * u).astype(jnp.bfloat16)
        part = jnp.dot(h, wd_ref[0], preferred_element_type=jnp.float32)

        @pl.when(j == 0)
        def _():
            o_ref[...] = part

        @pl.when(j > 0)
        def _():
            o_ref[...] = o_ref[...] + part

        @pl.when(j == NJ - 1)
        def _():
            o_ref[...] = o_ref[...] * tw_ref[0]


def _grouped_mlp(x_p, Wg, Wu, Wd, tw_p, be):
    grid_spec = pltpu.PrefetchScalarGridSpec(
        num_scalar_prefetch=1,
        grid=(MAXB, NJ),
        in_specs=[
            pl.BlockSpec((BT, D), lambda b, j, be: (b, 0)),
            pl.BlockSpec((1, D, BJ), lambda b, j, be: (be[b], 0, j)),
            pl.BlockSpec((1, D, BJ), lambda b, j, be: (be[b], 0, j)),
            pl.BlockSpec((1, BJ, D), lambda b, j, be: (be[b], j, 0)),
            pl.BlockSpec((1, BT, 1), lambda b, j, be: (b, 0, 0)),
        ],
        out_specs=pl.BlockSpec((BT, D), lambda b, j, be: (b, 0)),
    )
    return pl.pallas_call(
        _moe_body,
        grid_spec=grid_spec,
        out_shape=jax.ShapeDtypeStruct((PADN, D), jnp.float32),
    )(be, x_p, Wg, Wu, Wd, tw_p)


def kernel(hidden_states, router_W, router_b, Wg, Wu, Wd):
    B, S, _ = hidden_states.shape
    flat = hidden_states.reshape(T, D)

    tw, ti = _router(flat, router_W, router_b)
    topi = ti[:, 0]
    topw = tw[:, 0]

    order = jnp.argsort(topi, stable=True).astype(jnp.int32)
    counts = jnp.bincount(topi, length=E)
    nb = (counts + BT - 1) // BT
    cum_nb = jnp.cumsum(nb)
    pstart = (cum_nb - nb) * BT
    cstart = jnp.cumsum(counts) - counts
    e_sorted = topi[order]
    pos = (pstart[e_sorted] + jnp.arange(T) - cstart[e_sorted]).astype(jnp.int32)
    dest = jnp.zeros((T,), jnp.int32).at[order].set(pos)
    src = jnp.zeros((PADN,), jnp.int32).at[pos].set(order)
    bids = jnp.arange(MAXB)
    be = jnp.minimum(jnp.sum(bids[:, None] >= cum_nb[None, :], axis=1), E - 1)
    be = jnp.concatenate([be, cum_nb[-1:]]).astype(jnp.int32)

    x_p = flat.astype(jnp.bfloat16)[src]
    tw_p = topw[src].reshape(MAXB, BT, 1)

    y_p = _grouped_mlp(x_p, Wg.astype(jnp.bfloat16), Wu.astype(jnp.bfloat16),
                       Wd.astype(jnp.bfloat16), tw_p, be)
    out = y_p[dest]
    return out.reshape(B, S, D)
